# baseline (speedup 1.0000x reference)
import functools

import jax
import jax.numpy as jnp
from jax import lax
from jax.experimental import pallas as pl
from jax.experimental.pallas import tpu as pltpu
from jax.experimental.pallas import tpu_sc as plsc

L = 16
NC, NS = 2, 16
NW = NC * NS
BS = 16384
K, E, D = 16, 4, 64
B = 256
DH = D // 2
BLKS_PER_W = BS // (NW * B)
NG = B // L

_mesh = plsc.VectorSubcoreMesh(core_axis_name="c", subcore_axis_name="s")


@functools.partial(
    pl.kernel,
    out_type=(
        jax.ShapeDtypeStruct((D, BS), jnp.float32),
        jax.ShapeDtypeStruct((K, D, BS), jnp.float32),
    ),
    mesh=_mesh,
    compiler_params=pltpu.CompilerParams(needs_layout_passes=False),
    scratch_types=[
        pltpu.VMEM((2, E, DH, B), jnp.float32),
        pltpu.VMEM((2, DH, B), jnp.float32),
        pltpu.VMEM((2, DH, B), jnp.float32),
        pltpu.VMEM((2, E, 1, B), jnp.int32),
        pltpu.VMEM((DH, B), jnp.float32),
        pltpu.VMEM((D, B), jnp.float32),
        pltpu.SemaphoreType.DMA((2,)),
        pltpu.SemaphoreType.DMA((2,)),
        pltpu.SemaphoreType.DMA((2,)),
        pltpu.SemaphoreType.DMA,
        pltpu.SemaphoreType.DMA,
    ],
)
def _sc_agg(edge_hbm, ent_hbm, self_hbm, mask_hbm, sv_hbm, nbr_hbm,
            edge_v, ent_v, nbr_v, mask_v, self_v, sv_v,
            in_sem, out_sem, m_sem, f_sem, o_sem):
    wid = lax.axis_index("s") * NC + lax.axis_index("c")
    col0 = wid * (BLKS_PER_W * B)

    def start_in(k, dh, b0):
        pltpu.async_copy(edge_hbm.at[k, :, pl.ds(dh * DH, DH), pl.ds(b0, B)],
                         edge_v.at[dh], in_sem.at[dh])
        pltpu.async_copy(ent_hbm.at[k, pl.ds(dh * DH, DH), pl.ds(b0, B)],
                         ent_v.at[dh], in_sem.at[dh])

    def wait_in(k, dh, b0):
        pltpu.make_async_copy(
            edge_hbm.at[k, :, pl.ds(dh * DH, DH), pl.ds(b0, B)],
            edge_v.at[dh], in_sem.at[dh]).wait()
        pltpu.make_async_copy(
            ent_hbm.at[k, pl.ds(dh * DH, DH), pl.ds(b0, B)],
            ent_v.at[dh], in_sem.at[dh]).wait()

    def wait_out(dh):
        pltpu.make_async_copy(nbr_v.at[dh],
                              nbr_hbm.at[0, pl.ds(0, DH), pl.ds(0, B)],
                              out_sem.at[dh]).wait()

    def start_mask(k, par, b0):
        pltpu.async_copy(mask_hbm.at[0, k, :, :, pl.ds(b0, B)],
                         mask_v.at[par], m_sem.at[par])

    def wait_mask(par):
        pltpu.make_async_copy(mask_hbm.at[0, 0, :, :, pl.ds(0, B)],
                              mask_v.at[par], m_sem.at[par]).wait()

    start_mask(0, 0, col0)
    start_mask(1, 1, col0)
    start_in(0, 0, col0)

    @pl.loop(0, BLKS_PER_W)
    def _blk(blk):
        b0 = col0 + blk * B

        @pl.loop(0, D, unroll=2)
        def _z(d):
            for g in range(NG):
                sv_v[d, pl.ds(g * L, L)] = jnp.zeros((L,), jnp.float32)

        @pl.loop(0, K // 2)
        def _kk(kk):
            for half in range(2):
                k = 2 * kk + half
                for dh in range(2):
                    if dh == 0:
                        start_in(k, 1, b0)
                        wait_mask(half)
                    elif half == 0:
                        start_in(k + 1, 0, b0)
                    else:
                        @pl.when(kk < K // 2 - 1)
                        def _():
                            start_in(k + 1, 0, b0)

                        @pl.when((kk == K // 2 - 1) & (blk < BLKS_PER_W - 1))
                        def _():
                            start_in(0, 0, b0 + B)

                    wait_in(k, dh, b0)

                    if half == 0:
                        @pl.when(kk >= 1)
                        def _():
                            wait_out(dh)
                    else:
                        wait_out(dh)

                    for g in range(NG):
                        bb = g * L
                        ms = [mask_v[half, e, 0, pl.ds(bb, L)
                                     ].astype(jnp.float32) for e in range(E)]
                        cnt = (ms[0] + ms[1]) + (ms[2] + ms[3])
                        inv = 0.5 / jnp.maximum(cnt, 1.0)
                        cs = [m * inv for m in ms]

                        @plsc.parallel_loop(0, DH, unroll=4)
                        def _d(d, dh=dh, bb=bb, cs=cs):
                            ev = [edge_v[dh, e, d, pl.ds(bb, L)]
                                  for e in range(E)]
                            p01 = cs[0] * ev[0] + cs[1] * ev[1]
                            p23 = cs[2] * ev[2] + cs[3] * ev[3]
                            a = (ent_v[dh, d, pl.ds(bb, L)] + p01) + p23
                            nbr_v[dh, d, pl.ds(bb, L)] = a
                            sv_v[dh * DH + d, pl.ds(bb, L)] = (
                                sv_v[dh * DH + d, pl.ds(bb, L)] + a)

                    pltpu.async_copy(
                        nbr_v.at[dh],
                        nbr_hbm.at[k, pl.ds(dh * DH, DH), pl.ds(b0, B)],
                        out_sem.at[dh])

                    if dh == 1:
                        @pl.when(kk < K // 2 - 1)
                        def _():
                            start_mask(k + 2, half, b0)

                        @pl.when((kk == K // 2 - 1) & (blk < BLKS_PER_W - 1))
                        def _():
                            start_mask(half, half, b0 + B)

        for dh in range(2):
            pltpu.async_copy(self_hbm.at[pl.ds(dh * DH, DH), pl.ds(b0, B)],
                             self_v, f_sem)
            pltpu.make_async_copy(
                self_hbm.at[pl.ds(dh * DH, DH), pl.ds(b0, B)],
                self_v, f_sem).wait()

            @pl.loop(0, DH, unroll=2)
            def _f(d, dh=dh):
                for g in range(NG):
                    sv_v[dh * DH + d, pl.ds(g * L, L)] = (
                        self_v[d, pl.ds(g * L, L)]
                        + (0.5 / K) * sv_v[dh * DH + d, pl.ds(g * L, L)])

        pltpu.async_copy(sv_v, sv_hbm.at[:, pl.ds(b0, B)], o_sem)
        wait_out(0)
        wait_out(1)
        pltpu.make_async_copy(sv_v, sv_hbm.at[:, pl.ds(b0, B)], o_sem).wait()


def kernel(self_vectors, neighbor_entity_vectors, neighbor_edge_vectors, masks):
    edge_t = jnp.transpose(neighbor_edge_vectors, (1, 2, 3, 4, 0))[0]
    ent_t = jnp.transpose(neighbor_entity_vectors, (1, 2, 3, 0))[0]
    self_t = self_vectors.T
    mask_t = jnp.transpose(masks, (1, 2, 3, 4, 0))
    sv_t, nbr_t = _sc_agg(edge_t, ent_t, self_t, mask_t)
    sv = sv_t.T.reshape(BS, 1, D)
    nbr = jnp.transpose(nbr_t, (2, 0, 1)).reshape(BS, 1, K, D)
    return sv, nbr
